# baseline (speedup 1.0000x reference)
import functools

import jax
import jax.numpy as jnp
from jax import lax
from jax.experimental import pallas as pl
from jax.experimental.pallas import tpu as pltpu
from jax.experimental.pallas import tpu_sc as plsc

B = 8
C = 192
H = 224
W = 224
HW = H * W
HS = H // 2
SW = 256
K = HW // 4
NCHUNK = K // 16
NW = 32
CPB = NW // B
CPW = C // CPB


def _build():
    mesh = plsc.VectorSubcoreMesh(core_axis_name="c", subcore_axis_name="s")

    @functools.partial(
        pl.kernel,
        mesh=mesh,
        out_type=jax.ShapeDtypeStruct((B, C, K), jnp.float32),
        compiler_params=pltpu.CompilerParams(needs_layout_passes=False),
        scratch_types=[
            pltpu.VMEM((HS + 1, SW), jnp.float32),
            pltpu.VMEM((HS + 1, SW), jnp.float32),
            pltpu.VMEM((K + 16,), jnp.int32),
            pltpu.VMEM((K,), jnp.float32),
            pltpu.SemaphoreType.DMA,
            pltpu.SemaphoreType.DMA,
            pltpu.SemaphoreType.DMA,
        ],
    )
    def k(inp_hbm, mask_hbm, out_hbm, vbuf0, vbuf1, idxbuf, obuf,
          sem0, sem1, semo):
        cid = lax.axis_index("c")
        sid = lax.axis_index("s")
        wid = sid * 2 + cid
        b = wid // CPB
        ch0 = (wid % CPB) * CPW
        row0 = b * C + ch0
        iota = lax.iota(jnp.int32, 16)

        zrow = jnp.zeros((16,), jnp.float32)

        @plsc.parallel_loop(0, SW // 16, unroll=4)
        def _(t):
            vbuf0[HS, pl.ds(t * 16, 16)] = zrow
            vbuf1[HS, pl.ds(t * 16, 16)] = zrow

        pltpu.async_copy(
            mask_hbm.at[b, :, pl.ds(0, SW)], vbuf0.at[pl.ds(0, HS)], sem0
        ).wait()

        @plsc.parallel_loop(0, NCHUNK + 1, unroll=4)
        def _(t):
            idxbuf[pl.ds(t * 16, 16)] = jnp.full((16,), HS * SW, jnp.int32)

        def comp(rr, cnt):
            for t in range(W // 2 // 16):
                col = 32 * t + 2 * iota
                rsp = jnp.full((16,), rr, jnp.int32)
                mv = plsc.bitcast(
                    plsc.load_gather(vbuf0, [rsp, col]), jnp.int32)
                keep = mv >= 1
                k16 = keep.astype(jnp.int32)
                pos = cnt + plsc.cumsum(k16) - 1
                plsc.store_scatter(idxbuf, [pos], rr * SW + col, mask=keep)
                cnt = cnt + jnp.sum(k16)
            return cnt

        lax.fori_loop(0, HS, comp, jnp.int32(0))

        def gather_to(vbuf):
            @plsc.parallel_loop(0, NCHUNK, unroll=8)
            def _(j):
                off = j * 16
                v = idxbuf[pl.ds(off, 16)]
                row = lax.shift_right_logical(v, 8)
                col = lax.bitwise_and(v, 255)
                obuf[pl.ds(off, 16)] = plsc.load_gather(vbuf, [row, col])

        def start_out(chv):
            pltpu.async_copy(obuf, out_hbm.at[b, chv], semo)

        def wait_out():
            pltpu.make_async_copy(obuf, out_hbm.at[b, ch0], semo).wait()

        last_row = B * C - 1

        def start_in(r, vbuf, sem):
            rc = jnp.minimum(r, last_row)
            pltpu.async_copy(
                inp_hbm.at[rc, :, pl.ds(0, SW)], vbuf.at[pl.ds(0, HS)], sem)

        def wait_in(vbuf, sem):
            pltpu.make_async_copy(
                inp_hbm.at[0, :, pl.ds(0, SW)], vbuf.at[pl.ds(0, HS)], sem
            ).wait()

        start_in(row0, vbuf0, sem0)

        def chan(i, carry):
            wait_in(vbuf0, sem0)
            start_in(row0 + 2 * i + 1, vbuf1, sem1)

            @pl.when(i > 0)
            def _():
                wait_out()

            gather_to(vbuf0)
            start_out(ch0 + 2 * i)
            wait_in(vbuf1, sem1)
            start_in(row0 + 2 * i + 2, vbuf0, sem0)
            wait_out()
            gather_to(vbuf1)
            start_out(ch0 + 2 * i + 1)
            return carry

        lax.fori_loop(0, CPW // 2, chan, 0)
        wait_in(vbuf0, sem0)
        wait_out()

    return k


def kernel(input, pooling_mask):
    inp_rows = input.reshape(B * C, HS, 2 * W)
    mask_rows = lax.bitcast_convert_type(
        pooling_mask.reshape(B, HS, 2 * W), jnp.float32)
    return _build()(inp_rows, mask_rows)


# speedup vs baseline: 2.3362x; 2.3362x over previous
import functools

import jax
import jax.numpy as jnp
from jax import lax
from jax.experimental import pallas as pl
from jax.experimental.pallas import tpu as pltpu
from jax.experimental.pallas import tpu_sc as plsc

B = 8
C = 192
H = 224
W = 224
HW = H * W
K = HW // 4
NCHUNK = K // 16
NW = 32
CPB = NW // B
CPW = C // CPB


def _build():
    mesh = plsc.VectorSubcoreMesh(core_axis_name="c", subcore_axis_name="s")

    @functools.partial(
        pl.kernel,
        mesh=mesh,
        out_type=jax.ShapeDtypeStruct((B, C, K), jnp.float32),
        compiler_params=pltpu.CompilerParams(needs_layout_passes=False),
        scratch_types=[
            pltpu.VMEM((HW + 16,), jnp.float32),
            pltpu.VMEM((HW + 16,), jnp.float32),
            pltpu.VMEM((K + 16,), jnp.int32),
            pltpu.VMEM((K,), jnp.float32),
            pltpu.SemaphoreType.DMA,
            pltpu.SemaphoreType.DMA,
            pltpu.SemaphoreType.DMA,
        ],
    )
    def k(inp_hbm, mask_hbm, out_hbm, vbuf0, vbuf1, idxbuf, obuf,
          sem0, sem1, semo):
        cid = lax.axis_index("c")
        sid = lax.axis_index("s")
        wid = sid * 2 + cid
        b = wid // CPB
        ch0 = (wid % CPB) * CPW
        row0 = b * C + ch0
        iota = lax.iota(jnp.int32, 16)

        zrow = jnp.zeros((16,), jnp.float32)
        vbuf0[pl.ds(HW, 16)] = zrow
        vbuf1[pl.ds(HW, 16)] = zrow

        last_row = B * C - 1

        def start_in(r, vbuf, sem):
            pltpu.async_copy(
                inp_hbm.at[jnp.minimum(r, last_row)],
                vbuf.at[pl.ds(0, HW)], sem)

        def wait_in(vbuf, sem):
            pltpu.make_async_copy(
                inp_hbm.at[0], vbuf.at[pl.ds(0, HW)], sem).wait()

        pltpu.sync_copy(mask_hbm.at[b], vbuf0.at[pl.ds(0, HW)])
        start_in(row0, vbuf1, sem1)

        @plsc.parallel_loop(0, NCHUNK + 1, unroll=4)
        def _(t):
            idxbuf[pl.ds(t * 16, 16)] = jnp.full((16,), HW, jnp.int32)

        def comp(rr, cnt):
            for t in range(W // 2 // 16):
                fpos = 2 * W * rr + 32 * t + 2 * iota
                mv = plsc.bitcast(plsc.load_gather(vbuf0, [fpos]), jnp.int32)
                keep = mv >= 1
                cum = plsc.cumsum(keep.astype(jnp.int32))
                plsc.store_scatter(idxbuf, [cnt + cum - 1], fpos, mask=keep)
                cnt = cnt + cum[15]
            return cnt

        lax.fori_loop(0, H // 2, comp, jnp.int32(0))

        def gather_to(vbuf):
            @plsc.parallel_loop(0, NCHUNK, unroll=8)
            def _(j):
                off = j * 16
                v = idxbuf[pl.ds(off, 16)]
                obuf[pl.ds(off, 16)] = plsc.load_gather(vbuf, [v])

        def start_out(chv):
            pltpu.async_copy(obuf, out_hbm.at[b, chv], semo)

        def wait_out():
            pltpu.make_async_copy(obuf, out_hbm.at[b, ch0], semo).wait()

        def chan(i, carry):
            wait_in(vbuf1, sem1)
            start_in(row0 + 2 * i + 1, vbuf0, sem0)

            @pl.when(i > 0)
            def _():
                wait_out()

            gather_to(vbuf1)
            start_out(ch0 + 2 * i)
            wait_in(vbuf0, sem0)
            start_in(row0 + 2 * i + 2, vbuf1, sem1)
            wait_out()
            gather_to(vbuf0)
            start_out(ch0 + 2 * i + 1)
            return carry

        lax.fori_loop(0, CPW // 2, chan, 0)
        wait_in(vbuf1, sem1)
        wait_out()

    return k


def kernel(input, pooling_mask):
    inp_rows = input.reshape(B * C, HW)
    mask_rows = lax.bitcast_convert_type(
        pooling_mask.reshape(B, HW), jnp.float32)
    return _build()(inp_rows, mask_rows)
